# Initial kernel scaffold; baseline (speedup 1.0000x reference)
import functools

import jax
import jax.numpy as jnp
from jax import lax
from jax.experimental import pallas as pl
from jax.experimental.pallas import tpu as pltpu
from jax.experimental.pallas import tpu_sc as plsc

NC, NS = 2, 16
NW = NC * NS
CHUNK = 128
BN = 1024
BE = 2048
F32 = jnp.float32


def _cdiv(a, b):
    return (a + b - 1) // b


def _silu(v):
    return v * jax.nn.sigmoid(v)


def _dot(a, b):
    return jnp.dot(a, b, preferred_element_type=F32)


def _full_spec(shape):
    return pl.BlockSpec(shape, lambda i: (0,) * len(shape))


def _init_body(nf, x16, w1, b1, w2, b2, we, be, out):
    h = jax.nn.elu(_dot(nf[...], w1[...]) + b1[...])
    h = _dot(h, w2[...]) + b2[...]
    h = _dot(h, we[...]) + be[...]
    out[:, 0:32] = h
    out[:, 32:48] = x16[...]


def _edge_body(ehr, ehc, ea, w1a, w1b, w1r, w1e, eb1, w2, eb2, c1, cb1, c2,
               m_out, t_out):
    hr = ehr[:, 0:32]
    hc = ehc[:, 0:32]
    d = ehr[:, 32:48] - ehc[:, 32:48]
    radial = jnp.sum(d * d, axis=1, keepdims=True)
    t = (_dot(hr, w1a[...]) + _dot(hc, w1b[...]) + radial * w1r[...]
         + _dot(ea[...], w1e[...]) + eb1[...])
    m = _silu(_dot(_silu(t), w2[...]) + eb2[...])
    c = _silu(_dot(m, c1[...]) + cb1[...])
    s = jnp.sum(c * c2[...], axis=1, keepdims=True)
    m_out[...] = m
    t_out[...] = d[:, 0:4] * s + jnp.array([[0.0, 0.0, 0.0, 1.0]], dtype=F32)


def _node_mid_body(tin, am2, ax2, n1a, n1b, nb1, n2, nb2, out):
    h = tin[:, 0:32]
    x = tin[:, 32:48]
    am = am2[0] + am2[1]
    ax = ax2[0] + ax2[1]
    cnt = jnp.maximum(ax[:, 3:4], 1.0)
    delta = (ax / cnt) * jnp.array([[1.0, 1.0, 1.0, 0.0]], dtype=F32)
    delta16 = jnp.concatenate(
        [delta, jnp.zeros((delta.shape[0], 12), dtype=F32)], axis=1)
    hn = h + (_dot(_silu(_dot(h, n1a[...]) + _dot(am, n1b[...]) + nb1[...]),
                   n2[...]) + nb2[...])
    out[:, 0:32] = hn
    out[:, 32:48] = x + delta16


def _node_last_body(tin, am2, n1a, n1b, nb1, n2, nb2, wo, bo, out):
    h = tin[:, 0:32]
    am = am2[0] + am2[1]
    hn = h + (_dot(_silu(_dot(h, n1a[...]) + _dot(am, n1b[...]) + nb1[...]),
                   n2[...]) + nb2[...])
    out[...] = _dot(hn, wo[...]) + bo[...]


def _make_gather(n_pad, e_pad, cpw):
    mesh = plsc.VectorSubcoreMesh(core_axis_name="c", subcore_axis_name="s")

    @functools.partial(
        pl.kernel,
        out_type=(jax.ShapeDtypeStruct((e_pad, 48), F32),
                  jax.ShapeDtypeStruct((e_pad, 48), F32)),
        mesh=mesh,
        scratch_types=[
            pltpu.VMEM((cpw, CHUNK), jnp.int32),
            pltpu.VMEM((cpw, CHUNK), jnp.int32),
            pltpu.VMEM((CHUNK, 48), F32),
            pltpu.VMEM((CHUNK, 48), F32),
            pltpu.SemaphoreType.DMA,
            pltpu.SemaphoreType.DMA,
        ],
    )
    def gather(t_hbm, idxr_hbm, idxc_hbm, ehr_hbm, ehc_hbm,
               idxr_v, idxc_v, bufr, bufc, semr, semc):
        wid = lax.axis_index("s") * NC + lax.axis_index("c")
        base = wid * cpw * CHUNK
        pltpu.sync_copy(idxr_hbm.at[wid], idxr_v)
        pltpu.sync_copy(idxc_hbm.at[wid], idxc_v)

        def step(j, carry):
            cr = pltpu.async_copy(t_hbm.at[idxr_v.at[j]], bufr, semr)
            cc = pltpu.async_copy(t_hbm.at[idxc_v.at[j]], bufc, semc)
            cr.wait()
            cc.wait()
            pltpu.sync_copy(bufr, ehr_hbm.at[pl.ds(base + j * CHUNK, CHUNK)])
            pltpu.sync_copy(bufc, ehc_hbm.at[pl.ds(base + j * CHUNK, CHUNK)])
            return carry

        lax.fori_loop(0, cpw, step, 0)

    return gather


def _make_scatter(n_pad, e_pad, cpw):
    zpt = n_pad // NS
    mesh = plsc.VectorSubcoreMesh(core_axis_name="c", subcore_axis_name="s")

    @functools.partial(
        pl.kernel,
        out_type=(jax.ShapeDtypeStruct((NC * n_pad, 32), F32),
                  jax.ShapeDtypeStruct((NC * n_pad, 4), F32)),
        mesh=mesh,
        scratch_types=[
            pltpu.VMEM_SHARED((n_pad, 32), F32),
            pltpu.VMEM_SHARED((n_pad, 4), F32),
            pltpu.VMEM((cpw, CHUNK), jnp.int32),
            pltpu.VMEM((CHUNK, 32), F32),
            pltpu.VMEM((CHUNK, 4), F32),
        ],
    )
    def scatter(m_hbm, tr_hbm, idxr_hbm, zm_hbm, zx_hbm, aggm_hbm, aggx_hbm,
                spm, spx, idx_v, mbuf, tbuf):
        cid = lax.axis_index("c")
        sid = lax.axis_index("s")
        wid = sid * NC + cid
        base = wid * cpw * CHUNK
        pltpu.sync_copy(zm_hbm, spm.at[pl.ds(sid * zpt, zpt)])
        pltpu.sync_copy(zx_hbm, spx.at[pl.ds(sid * zpt, zpt)])
        pltpu.sync_copy(idxr_hbm.at[wid], idx_v)
        plsc.subcore_barrier()

        def step(j, carry):
            pltpu.sync_copy(m_hbm.at[pl.ds(base + j * CHUNK, CHUNK)], mbuf)
            pltpu.sync_copy(tr_hbm.at[pl.ds(base + j * CHUNK, CHUNK)], tbuf)
            pltpu.sync_copy(mbuf, spm.at[idx_v.at[j]], add=True)
            pltpu.sync_copy(tbuf, spx.at[idx_v.at[j]], add=True)
            return carry

        lax.fori_loop(0, cpw, step, 0)
        plsc.subcore_barrier()
        off = cid * n_pad + sid * zpt
        pltpu.sync_copy(spm.at[pl.ds(sid * zpt, zpt)],
                        aggm_hbm.at[pl.ds(off, zpt)])
        pltpu.sync_copy(spx.at[pl.ds(sid * zpt, zpt)],
                        aggx_hbm.at[pl.ds(off, zpt)])

    return scatter


def kernel(node_feat, coords, edge_index, edge_attr, params):
    n, d_in = node_feat.shape
    e = edge_index.shape[1]
    e_pad = _cdiv(e, NW * CHUNK) * (NW * CHUNK)
    cpw = e_pad // (NW * CHUNK)
    n_pad = _cdiv(n + 1, NS * CHUNK) * (NS * CHUNK)
    zpt = n_pad // NS
    grid_n = n_pad // BN
    grid_e = e_pad // BE
    d_pad = _cdiv(d_in, 8) * 8

    nf = jnp.zeros((n_pad, d_pad), F32).at[:n, :d_in].set(node_feat)
    x16 = jnp.zeros((n_pad, 16), F32).at[:n, :3].set(coords)
    pad_idx = jnp.full((e_pad - e,), n, jnp.int32)
    row = jnp.concatenate([edge_index[0], pad_idx]).reshape(NW, cpw, CHUNK)
    col = jnp.concatenate([edge_index[1], pad_idx]).reshape(NW, cpw, CHUNK)
    ea = jnp.zeros((e_pad, 8), F32).at[:e, :2].set(edge_attr)
    zm = jnp.zeros((zpt, 32), F32)
    zx = jnp.zeros((zpt, 4), F32)

    p = params
    w1 = jnp.zeros((d_pad, d_pad), F32).at[:d_in, :d_in].set(p["lin1"]["W"])
    b1 = jnp.zeros((1, d_pad), F32).at[0, :d_in].set(p["lin1"]["b"])
    w2 = jnp.zeros((d_pad, 32), F32).at[:d_in].set(p["lin2"]["W"])
    b2 = p["lin2"]["b"].reshape(1, 32)
    we = p["emb_in"]["W"]
    be = p["emb_in"]["b"].reshape(1, 32)

    tcur = pl.pallas_call(
        _init_body,
        grid=(grid_n,),
        in_specs=[
            pl.BlockSpec((BN, d_pad), lambda i: (i, 0)),
            pl.BlockSpec((BN, 16), lambda i: (i, 0)),
            _full_spec((d_pad, d_pad)), _full_spec((1, d_pad)),
            _full_spec((d_pad, 32)), _full_spec((1, 32)),
            _full_spec((32, 32)), _full_spec((1, 32)),
        ],
        out_specs=pl.BlockSpec((BN, 48), lambda i: (i, 0)),
        out_shape=jax.ShapeDtypeStruct((n_pad, 48), F32),
    )(nf, x16, w1, b1, w2, b2, we, be)

    sc_gather = _make_gather(n_pad, e_pad, cpw)
    sc_scatter = _make_scatter(n_pad, e_pad, cpw)

    n_layers = len(p["layers"])
    out = None
    for li, lp in enumerate(p["layers"]):
        ehr, ehc = sc_gather(tcur, row, col)

        w1full = lp["edge1"]["W"]
        w1a = w1full[0:32]
        w1b = w1full[32:64]
        w1r = w1full[64:65]
        w1e = jnp.zeros((8, 32), F32).at[:2].set(w1full[65:67])
        eb1 = lp["edge1"]["b"].reshape(1, 32)
        ew2 = lp["edge2"]["W"]
        eb2 = lp["edge2"]["b"].reshape(1, 32)
        c1 = lp["coord1"]["W"]
        cb1 = lp["coord1"]["b"].reshape(1, 32)
        c2 = lp["coord2"]["W"].reshape(1, 32)

        m, tr = pl.pallas_call(
            _edge_body,
            grid=(grid_e,),
            in_specs=[
                pl.BlockSpec((BE, 48), lambda i: (i, 0)),
                pl.BlockSpec((BE, 48), lambda i: (i, 0)),
                pl.BlockSpec((BE, 8), lambda i: (i, 0)),
                _full_spec((32, 32)), _full_spec((32, 32)),
                _full_spec((1, 32)), _full_spec((8, 32)),
                _full_spec((1, 32)), _full_spec((32, 32)),
                _full_spec((1, 32)), _full_spec((32, 32)),
                _full_spec((1, 32)), _full_spec((1, 32)),
            ],
            out_specs=[
                pl.BlockSpec((BE, 32), lambda i: (i, 0)),
                pl.BlockSpec((BE, 4), lambda i: (i, 0)),
            ],
            out_shape=[
                jax.ShapeDtypeStruct((e_pad, 32), F32),
                jax.ShapeDtypeStruct((e_pad, 4), F32),
            ],
        )(ehr, ehc, ea, w1a, w1b, w1r, w1e, eb1, ew2, eb2, c1, cb1, c2)

        aggm, aggx = sc_scatter(m, tr, row, zm, zx)
        am2 = aggm.reshape(NC, n_pad, 32)
        ax2 = aggx.reshape(NC, n_pad, 4)

        n1full = lp["node1"]["W"]
        n1a = n1full[0:32]
        n1b = n1full[32:64]
        nb1 = lp["node1"]["b"].reshape(1, 32)
        n2 = lp["node2"]["W"]
        nb2 = lp["node2"]["b"].reshape(1, 32)

        if li < n_layers - 1:
            tcur = pl.pallas_call(
                _node_mid_body,
                grid=(grid_n,),
                in_specs=[
                    pl.BlockSpec((BN, 48), lambda i: (i, 0)),
                    pl.BlockSpec((NC, BN, 32), lambda i: (0, i, 0)),
                    pl.BlockSpec((NC, BN, 4), lambda i: (0, i, 0)),
                    _full_spec((32, 32)), _full_spec((32, 32)),
                    _full_spec((1, 32)), _full_spec((32, 32)),
                    _full_spec((1, 32)),
                ],
                out_specs=pl.BlockSpec((BN, 48), lambda i: (i, 0)),
                out_shape=jax.ShapeDtypeStruct((n_pad, 48), F32),
            )(tcur, am2, ax2, n1a, n1b, nb1, n2, nb2)
        else:
            wo = p["emb_out"]["W"]
            bo = p["emb_out"]["b"].reshape(1, 32)
            out = pl.pallas_call(
                _node_last_body,
                grid=(grid_n,),
                in_specs=[
                    pl.BlockSpec((BN, 48), lambda i: (i, 0)),
                    pl.BlockSpec((NC, BN, 32), lambda i: (0, i, 0)),
                    _full_spec((32, 32)), _full_spec((32, 32)),
                    _full_spec((1, 32)), _full_spec((32, 32)),
                    _full_spec((1, 32)), _full_spec((32, 32)),
                    _full_spec((1, 32)),
                ],
                out_specs=pl.BlockSpec((BN, 32), lambda i: (i, 0)),
                out_shape=jax.ShapeDtypeStruct((n_pad, 32), F32),
            )(tcur, am2, n1a, n1b, nb1, n2, nb2, wo, bo)

    return out[:n]


# speedup vs baseline: 4.4232x; 4.4232x over previous
import functools

import jax
import jax.numpy as jnp
from jax import lax
from jax.experimental import pallas as pl
from jax.experimental.pallas import tpu as pltpu
from jax.experimental.pallas import tpu_sc as plsc

NC, NS = 2, 16
NW = NC * NS
CHUNK = 128
BN = 1024
BE = 2048
F32 = jnp.float32


def _cdiv(a, b):
    return (a + b - 1) // b


def _silu(v):
    return v * jax.nn.sigmoid(v)


def _bf(v):
    return v.astype(jnp.bfloat16)


def _dot(a, b):
    return jnp.dot(_bf(a), _bf(b), preferred_element_type=F32)


def _full_spec(shape):
    return pl.BlockSpec(shape, lambda i: (0,) * len(shape))


def _init_body(nf, x16, w1, b1, w2, b2, we, be, out):
    z = _dot(nf[...], w1[...]) + b1[...]
    h = jnp.where(z > 0, z, jnp.exp(jnp.minimum(z, 0.0)) - 1.0)
    h = _dot(h, w2[...]) + b2[...]
    h = _dot(h, we[...]) + be[...]
    out[:, 0:32] = h
    out[:, 32:48] = x16[...]


def _edge_body(ehr, ehc, ea, w1a, w1b, w1r, w1e, eb1, w2, eb2, c1, cb1, c2,
               p_out):
    hr = ehr[:, 0:32]
    hc = ehc[:, 0:32]
    d = ehr[:, 32:48] - ehc[:, 32:48]
    radial = jnp.sum(d * d, axis=1, keepdims=True)
    t = (_dot(hr, w1a[...]) + _dot(hc, w1b[...])
         + _bf(radial).astype(F32) * _bf(w1r[...]).astype(F32)
         + _dot(ea[...], w1e[...]) + eb1[...])
    m = _silu(_dot(_silu(t), w2[...]) + eb2[...])
    c = _silu(_dot(m, c1[...]) + cb1[...])
    s = jnp.sum(_bf(c).astype(F32) * _bf(c2[...]).astype(F32),
                axis=1, keepdims=True)
    p_out[:, 0:32] = m
    lane8 = lax.broadcasted_iota(jnp.int32, (1, 8), 1)
    p_out[:, 32:40] = d[:, 0:8] * s + (lane8 == 3).astype(F32)


def _node_mid_body(tin, am2, ax2, n1a, n1b, nb1, n2, nb2, out):
    h = tin[:, 0:32]
    x = tin[:, 32:48]
    am = am2[0, :, 0:32] + am2[1, :, 0:32]
    ax = ax2[0, :, 0:8] + ax2[1, :, 0:8]
    cnt = jnp.maximum(ax[:, 3:4], 1.0)
    lane16 = lax.broadcasted_iota(jnp.int32, (1, 16), 1)
    delta16 = (jnp.pad(ax / cnt, ((0, 0), (0, 8)))
               * (lane16 < 3).astype(F32))
    hn = h + (_dot(_silu(_dot(h, n1a[...]) + _dot(am, n1b[...]) + nb1[...]),
                   n2[...]) + nb2[...])
    out[:, 0:32] = hn
    out[:, 32:48] = x + delta16


def _node_last_body(tin, am2, n1a, n1b, nb1, n2, nb2, wo, bo, out):
    h = tin[:, 0:32]
    am = am2[0, :, 0:32] + am2[1, :, 0:32]
    hn = h + (_dot(_silu(_dot(h, n1a[...]) + _dot(am, n1b[...]) + nb1[...]),
                   n2[...]) + nb2[...])
    out[...] = _dot(hn, wo[...]) + bo[...]


def _make_gather(n_pad, e_pad, cpw):
    K = 4
    GB = 7
    assert cpw % (K * GB) == 0
    ngrp = cpw // (K * GB)
    nb = cpw // K
    mesh = plsc.VectorSubcoreMesh(core_axis_name="c", subcore_axis_name="s")

    @functools.partial(
        pl.kernel,
        out_type=(jax.ShapeDtypeStruct((e_pad, 128), F32),
                  jax.ShapeDtypeStruct((e_pad, 128), F32)),
        mesh=mesh,
        scratch_types=[
            pltpu.VMEM((GB * K, CHUNK), jnp.int32),
            pltpu.VMEM((GB * K, CHUNK), jnp.int32),
            pltpu.VMEM((2, K * CHUNK, 48), F32),
            pltpu.VMEM((2, K * CHUNK, 48), F32),
            pltpu.SemaphoreType.DMA,
            pltpu.SemaphoreType.DMA,
        ],
        compiler_params=pltpu.CompilerParams(use_tc_tiling_on_sc=False),
    )
    def gather(t_hbm, idxr_hbm, idxc_hbm, ehr_hbm, ehc_hbm,
               idxr_v, idxc_v, bufr, bufc, semg, semw):
        wid = lax.axis_index("s") * NC + lax.axis_index("c")
        base = wid * cpw * CHUNK

        def wr(b, bank):
            off = base + b * K * CHUNK
            return (pltpu.make_async_copy(
                        bufr.at[bank],
                        ehr_hbm.at[pl.ds(off, K * CHUNK), pl.ds(0, 48)],
                        semw),
                    pltpu.make_async_copy(
                        bufc.at[bank],
                        ehc_hbm.at[pl.ds(off, K * CHUNK), pl.ds(0, 48)],
                        semw))

        def grp_body(g, carry):
            pltpu.sync_copy(idxr_hbm.at[pl.ds(wid * cpw + g * GB * K, GB * K)],
                            idxr_v)
            pltpu.sync_copy(idxc_hbm.at[pl.ds(wid * cpw + g * GB * K, GB * K)],
                            idxc_v)

            def burst(t, carry2):
                b = g * GB + t
                bank = lax.rem(b, 2)

                @pl.when(b >= 1)
                def _():
                    w0, w1 = wr(b - 1, 1 - bank)
                    w0.wait()
                    w1.wait()

                gs = []
                for k in range(K):
                    gs.append(pltpu.make_async_copy(
                        t_hbm.at[idxr_v.at[t * K + k]],
                        bufr.at[bank, pl.ds(k * CHUNK, CHUNK)], semg))
                    gs.append(pltpu.make_async_copy(
                        t_hbm.at[idxc_v.at[t * K + k]],
                        bufc.at[bank, pl.ds(k * CHUNK, CHUNK)], semg))
                for cp in gs:
                    cp.start()
                for cp in gs:
                    cp.wait()
                w0, w1 = wr(b, bank)
                w0.start()
                w1.start()
                return carry2

            lax.fori_loop(0, GB, burst, 0)
            return carry

        lax.fori_loop(0, ngrp, grp_body, 0)
        w0, w1 = wr(nb - 1, (nb - 1) % 2)
        w0.wait()
        w1.wait()

    return gather


def _make_scatter(n_pad, e_pad, cpw, grp, lane0, width):
    zpt = n_pad // NS
    assert cpw % grp == 0
    mesh = plsc.VectorSubcoreMesh(core_axis_name="c", subcore_axis_name="s")

    @functools.partial(
        pl.kernel,
        out_type=jax.ShapeDtypeStruct((NC * n_pad, 128), F32),
        mesh=mesh,
        scratch_types=[
            pltpu.VMEM_SHARED((n_pad, width), F32),
            pltpu.VMEM((grp, CHUNK), jnp.int32),
            pltpu.VMEM((2, CHUNK, width), F32),
            pltpu.SemaphoreType.DMA,
        ],
        compiler_params=pltpu.CompilerParams(use_tc_tiling_on_sc=False),
    )
    def scatter(m_hbm, idxr_hbm, zm_hbm, aggm_hbm, spm, idx_v, mbuf, seml):
        cid = lax.axis_index("c")
        sid = lax.axis_index("s")
        wid = sid * NC + cid
        base = wid * cpw * CHUNK
        pltpu.sync_copy(zm_hbm.at[:, pl.ds(0, width)],
                        spm.at[pl.ds(sid * zpt, zpt)])
        plsc.subcore_barrier()

        def load(jj, slot):
            return pltpu.make_async_copy(
                m_hbm.at[pl.ds(base + jj * CHUNK, CHUNK),
                         pl.ds(lane0, width)],
                mbuf.at[slot], seml)

        def group(g, carry):
            pltpu.sync_copy(idxr_hbm.at[pl.ds(wid * cpw + g * grp, grp)],
                            idx_v)

            @pl.when(g == 0)
            def _():
                load(0, 0).start()

            def step(j, carry2):
                jj = g * grp + j
                slot = lax.rem(jj, 2)

                @pl.when(jj + 1 < cpw)
                def _():
                    load(jj + 1, 1 - slot).start()

                load(jj, slot).wait()
                pltpu.sync_copy(mbuf.at[slot], spm.at[idx_v.at[j]], add=True)
                return carry2

            lax.fori_loop(0, grp, step, 0)
            return carry

        lax.fori_loop(0, cpw // grp, group, 0)
        plsc.subcore_barrier()
        off = cid * n_pad + sid * zpt
        pltpu.sync_copy(spm.at[pl.ds(sid * zpt, zpt)],
                        aggm_hbm.at[pl.ds(off, zpt), pl.ds(0, width)])

    return scatter


def kernel(node_feat, coords, edge_index, edge_attr, params):
    n, d_in = node_feat.shape
    e = edge_index.shape[1]
    e_pad = _cdiv(e, NW * CHUNK) * (NW * CHUNK)
    cpw = e_pad // (NW * CHUNK)
    n_pad = _cdiv(n + 1, NS * CHUNK) * (NS * CHUNK)
    zpt = n_pad // NS
    grid_n = n_pad // BN
    grid_e = e_pad // BE
    d_pad = _cdiv(d_in, 8) * 8

    nf = jnp.zeros((n_pad, d_pad), F32).at[:n, :d_in].set(node_feat)
    x16 = jnp.zeros((n_pad, 16), F32).at[:n, :3].set(coords)
    pad_idx = jnp.full((e_pad - e,), n, jnp.int32)
    row = jnp.concatenate([edge_index[0], pad_idx]).reshape(NW * cpw, CHUNK)
    col = jnp.concatenate([edge_index[1], pad_idx]).reshape(NW * cpw, CHUNK)
    ea = jnp.zeros((e_pad, 8), F32).at[:e, :2].set(edge_attr)
    z128 = jnp.zeros((zpt, 128), F32)

    p = params
    w1 = jnp.zeros((d_pad, d_pad), F32).at[:d_in, :d_in].set(p["lin1"]["W"])
    b1 = jnp.zeros((1, d_pad), F32).at[0, :d_in].set(p["lin1"]["b"])
    w2 = jnp.zeros((d_pad, 32), F32).at[:d_in].set(p["lin2"]["W"])
    b2 = p["lin2"]["b"].reshape(1, 32)
    we = p["emb_in"]["W"]
    be = p["emb_in"]["b"].reshape(1, 32)

    tcur = pl.pallas_call(
        _init_body,
        grid=(grid_n,),
        in_specs=[
            pl.BlockSpec((BN, d_pad), lambda i: (i, 0)),
            pl.BlockSpec((BN, 16), lambda i: (i, 0)),
            _full_spec((d_pad, d_pad)), _full_spec((1, d_pad)),
            _full_spec((d_pad, 32)), _full_spec((1, 32)),
            _full_spec((32, 32)), _full_spec((1, 32)),
        ],
        out_specs=pl.BlockSpec((BN, 48), lambda i: (i, 0)),
        out_shape=jax.ShapeDtypeStruct((n_pad, 48), F32),
    )(nf, x16, w1, b1, w2, b2, we, be)

    sc_gather = _make_gather(n_pad, e_pad, cpw)
    scat_grp = max(g for g in range(1, cpw + 1) if cpw % g == 0 and g <= 49)
    sc_scatter_m = _make_scatter(n_pad, e_pad, cpw, scat_grp, 0, 32)
    sc_scatter_x = _make_scatter(n_pad, e_pad, cpw, scat_grp, 32, 8)

    n_layers = len(p["layers"])
    out = None
    for li, lp in enumerate(p["layers"]):
        ehr, ehc = sc_gather(tcur, row, col)

        w1full = lp["edge1"]["W"]
        w1a = w1full[0:32]
        w1b = w1full[32:64]
        w1r = w1full[64:65]
        w1e = jnp.zeros((8, 32), F32).at[:2].set(w1full[65:67])
        eb1 = lp["edge1"]["b"].reshape(1, 32)
        ew2 = lp["edge2"]["W"]
        eb2 = lp["edge2"]["b"].reshape(1, 32)
        c1 = lp["coord1"]["W"]
        cb1 = lp["coord1"]["b"].reshape(1, 32)
        c2 = lp["coord2"]["W"].reshape(1, 32)

        payload = pl.pallas_call(
            _edge_body,
            grid=(grid_e,),
            in_specs=[
                pl.BlockSpec((BE, 128), lambda i: (i, 0)),
                pl.BlockSpec((BE, 128), lambda i: (i, 0)),
                pl.BlockSpec((BE, 8), lambda i: (i, 0)),
                _full_spec((32, 32)), _full_spec((32, 32)),
                _full_spec((1, 32)), _full_spec((8, 32)),
                _full_spec((1, 32)), _full_spec((32, 32)),
                _full_spec((1, 32)), _full_spec((32, 32)),
                _full_spec((1, 32)), _full_spec((1, 32)),
            ],
            out_specs=pl.BlockSpec((BE, 128), lambda i: (i, 0)),
            out_shape=jax.ShapeDtypeStruct((e_pad, 128), F32),
        )(ehr, ehc, ea, w1a, w1b, w1r, w1e, eb1, ew2, eb2, c1, cb1, c2)

        aggm = sc_scatter_m(payload, row, z128)
        aggx = sc_scatter_x(payload, row, z128)
        am2 = aggm.reshape(NC, n_pad, 128)
        ax2 = aggx.reshape(NC, n_pad, 128)

        n1full = lp["node1"]["W"]
        n1a = n1full[0:32]
        n1b = n1full[32:64]
        nb1 = lp["node1"]["b"].reshape(1, 32)
        n2 = lp["node2"]["W"]
        nb2 = lp["node2"]["b"].reshape(1, 32)

        if li < n_layers - 1:
            tcur = pl.pallas_call(
                _node_mid_body,
                grid=(grid_n,),
                in_specs=[
                    pl.BlockSpec((BN, 48), lambda i: (i, 0)),
                    pl.BlockSpec((NC, BN, 128), lambda i: (0, i, 0)),
                    pl.BlockSpec((NC, BN, 128), lambda i: (0, i, 0)),
                    _full_spec((32, 32)), _full_spec((32, 32)),
                    _full_spec((1, 32)), _full_spec((32, 32)),
                    _full_spec((1, 32)),
                ],
                out_specs=pl.BlockSpec((BN, 48), lambda i: (i, 0)),
                out_shape=jax.ShapeDtypeStruct((n_pad, 48), F32),
            )(tcur, am2, ax2, n1a, n1b, nb1, n2, nb2)
        else:
            wo = p["emb_out"]["W"]
            bo = p["emb_out"]["b"].reshape(1, 32)
            out = pl.pallas_call(
                _node_last_body,
                grid=(grid_n,),
                in_specs=[
                    pl.BlockSpec((BN, 48), lambda i: (i, 0)),
                    pl.BlockSpec((NC, BN, 128), lambda i: (0, i, 0)),
                    _full_spec((32, 32)), _full_spec((32, 32)),
                    _full_spec((1, 32)), _full_spec((32, 32)),
                    _full_spec((1, 32)), _full_spec((32, 32)),
                    _full_spec((1, 32)),
                ],
                out_specs=pl.BlockSpec((BN, 32), lambda i: (i, 0)),
                out_shape=jax.ShapeDtypeStruct((n_pad, 32), F32),
            )(tcur, am2, n1a, n1b, nb1, n2, nb2, wo, bo)

    return out[:n]
